# baseline (speedup 1.0000x reference)
import functools

import jax
import jax.numpy as jnp
from jax import lax
from jax.experimental import pallas as pl
from jax.experimental.pallas import tpu as pltpu
from jax.experimental.pallas import tpu_sc as plsc

U_NUM = 5000
I_NUM = 5000
N_NODES = U_NUM + I_NUM
DIM = 128
HALF = 64
LAYERS = 3
N_EDGES = 320000

NC = 2
NS = 16
GRP = 128

N_PAD = 10240
E_PAD = 327680
GROUPS = E_PAD // (NS * GRP)
RPT = N_PAD // NS
RSUB = 32
NSUB = RPT // RSUB


def _sc_body(h0, h1, srcr, dstr, valr,
             out0, out1,
             spa, spb, valv, rows0, rows1, bufa, bufb,
             srcs, dsts,
             semg0, semg1, sems0, sems1, semi0, semi1):
    c = lax.axis_index("c")
    s = lax.axis_index("s")
    src0, src1 = srcs.at[0], srcs.at[1]
    dst0, dst1 = dsts.at[0], dsts.at[1]

    pltpu.sync_copy(valr.at[s], valv)

    zero16 = jnp.zeros((16,), jnp.float32)

    def zero_buf(buf, nrows):
        @pl.loop(0, nrows)
        def _(i):
            for q in range(4):
                buf[i, pl.ds(q * 16, 16)] = zero16

    zero_buf(bufb, RSUB)

    @pl.loop(0, NSUB)
    def _(r):
        sl = pl.ds(s * RPT + r * RSUB, RSUB)

        @pl.when(c == 0)
        def _():
            pltpu.sync_copy(h0.at[sl], bufa)
            pltpu.sync_copy(bufa, out0.at[sl])

        @pl.when(c == 1)
        def _():
            pltpu.sync_copy(h1.at[sl], bufa)
            pltpu.sync_copy(bufa, out1.at[sl])

        pltpu.sync_copy(bufa, spa.at[sl])
        pltpu.sync_copy(bufb, spb.at[sl])

    plsc.subcore_barrier()

    def stage_idx(gg, sbuf, dbuf, sm):
        pltpu.async_copy(srcr.at[s].at[gg], sbuf, sm)
        pltpu.async_copy(dstr.at[s].at[gg], dbuf, sm)

    def idx_wait(sbuf, dbuf, sm):
        pltpu.make_async_copy(srcr.at[s].at[0], sbuf, sm).wait()
        pltpu.make_async_copy(dstr.at[s].at[0], dbuf, sm).wait()

    def scale(buf, gg):
        gbase = gg * GRP

        @plsc.parallel_loop(0, GRP, unroll=8)
        def _(i):
            sp = plsc.load_gather(valv, [jnp.full((16,), gbase + i, jnp.int32)])
            for q in range(4):
                buf[i, pl.ds(q * 16, 16)] = buf[i, pl.ds(q * 16, 16)] * sp

    for l in range(LAYERS):
        last = l == LAYERS - 1
        srcsp = spa if l % 2 == 0 else spb
        dstsp = spb if l % 2 == 0 else spa

        def gather_start(buf, sbuf, sm):
            pltpu.async_copy(srcsp.at[sbuf], buf, sm)

        def gather_wait(buf, sbuf, sm):
            pltpu.make_async_copy(srcsp.at[sbuf], buf, sm).wait()

        def scatter_start(buf, dbuf, sm):
            pltpu.async_copy(buf, dstsp.at[dbuf], sm, add=True)

        def scatter_wait(buf, dbuf, sm):
            pltpu.make_async_copy(buf, dstsp.at[dbuf], sm).wait()

        stage_idx(0, src0, dst0, semi0)
        idx_wait(src0, dst0, semi0)
        gather_start(rows0, src0, semg0)

        @pl.loop(0, GROUPS, step=2)
        def _(g):
            @pl.when(g > 0)
            def _():
                scatter_wait(rows1, dst1, sems1)
            stage_idx(g + 1, src1, dst1, semi1)
            idx_wait(src1, dst1, semi1)
            gather_start(rows1, src1, semg1)
            gather_wait(rows0, src0, semg0)
            scale(rows0, g)
            scatter_start(rows0, dst0, sems0)

            gather_wait(rows1, src1, semg1)
            scatter_wait(rows0, dst0, sems0)

            @pl.when(g + 2 < GROUPS)
            def _():
                stage_idx(g + 2, src0, dst0, semi0)
                idx_wait(src0, dst0, semi0)
                gather_start(rows0, src0, semg0)

            scale(rows1, g + 1)
            scatter_start(rows1, dst1, sems1)

        scatter_wait(rows1, dst1, sems1)
        plsc.subcore_barrier()

        @pl.loop(0, NSUB)
        def _(r):
            sl = pl.ds(s * RPT + r * RSUB, RSUB)
            pltpu.sync_copy(dstsp.at[sl], bufa)

            @pl.when(c == 0)
            def _():
                pltpu.sync_copy(out0.at[sl], bufb)

            @pl.when(c == 1)
            def _():
                pltpu.sync_copy(out1.at[sl], bufb)

            @pl.loop(0, RSUB)
            def _(i):
                for q in range(4):
                    v = bufa[i, pl.ds(q * 16, 16)] + bufb[i, pl.ds(q * 16, 16)]
                    if last:
                        v = v * jnp.float32(1.0 / (LAYERS + 1))
                    bufb[i, pl.ds(q * 16, 16)] = v

            @pl.when(c == 0)
            def _():
                pltpu.sync_copy(bufb, out0.at[sl])

            @pl.when(c == 1)
            def _():
                pltpu.sync_copy(bufb, out1.at[sl])

            if not last:
                zero_buf(bufa, RSUB)
                pltpu.sync_copy(bufa, srcsp.at[sl])

        if not last:
            plsc.subcore_barrier()


@functools.partial(
    pl.kernel,
    out_type=(
        jax.ShapeDtypeStruct((N_PAD, HALF), jnp.float32),
        jax.ShapeDtypeStruct((N_PAD, HALF), jnp.float32),
    ),
    mesh=plsc.VectorSubcoreMesh(
        core_axis_name="c", subcore_axis_name="s", num_cores=NC, num_subcores=NS
    ),
    compiler_params=pltpu.CompilerParams(
        needs_layout_passes=False, use_tc_tiling_on_sc=False
    ),
    scratch_types=[
        pltpu.VMEM_SHARED((N_PAD, HALF), jnp.float32),
        pltpu.VMEM_SHARED((N_PAD, HALF), jnp.float32),
        pltpu.VMEM((GROUPS * GRP,), jnp.float32),
        pltpu.VMEM((GRP, HALF), jnp.float32),
        pltpu.VMEM((GRP, HALF), jnp.float32),
        pltpu.VMEM((RSUB, HALF), jnp.float32),
        pltpu.VMEM((RSUB, HALF), jnp.float32),
        pltpu.VMEM((2, GRP), jnp.int32),
        pltpu.VMEM((2, GRP), jnp.int32),
        pltpu.SemaphoreType.DMA,
        pltpu.SemaphoreType.DMA,
        pltpu.SemaphoreType.DMA,
        pltpu.SemaphoreType.DMA,
        pltpu.SemaphoreType.DMA,
        pltpu.SemaphoreType.DMA,
    ],
)
def _lightgcn_sc(h0, h1, srcr, dstr, valr, out0, out1,
                 spa, spb, valv, rows0, rows1, bufa, bufb,
                 srcs, dsts,
                 semg0, semg1, sems0, sems1, semi0, semi1):
    _sc_body(h0, h1, srcr, dstr, valr, out0, out1,
             spa, spb, valv, rows0, rows1, bufa, bufb,
             srcs, dsts,
             semg0, semg1, sems0, sems1, semi0, semi1)


def kernel(user_embeds, item_embeds, adj_values, adj_indices, keep_rate):
    del keep_rate
    f32 = jnp.float32

    h0 = jnp.zeros((N_PAD, HALF), f32)
    h0 = h0.at[:U_NUM].set(user_embeds[:, :HALF].astype(f32))
    h0 = h0.at[U_NUM:N_NODES].set(item_embeds[:, :HALF].astype(f32))
    h1 = jnp.zeros((N_PAD, HALF), f32)
    h1 = h1.at[:U_NUM].set(user_embeds[:, HALF:].astype(f32))
    h1 = h1.at[U_NUM:N_NODES].set(item_embeds[:, HALF:].astype(f32))

    pad = E_PAD - N_EDGES
    src = jnp.concatenate(
        [adj_indices[1].astype(jnp.int32), jnp.zeros((pad,), jnp.int32)]
    ).reshape(NS, GROUPS, GRP)
    dst = jnp.concatenate(
        [adj_indices[0].astype(jnp.int32), jnp.zeros((pad,), jnp.int32)]
    ).reshape(NS, GROUPS, GRP)
    val = jnp.concatenate(
        [adj_values.astype(f32), jnp.zeros((pad,), f32)]
    ).reshape(NS, GROUPS * GRP)

    out0, out1 = _lightgcn_sc(h0, h1, src, dst, val)
    final = jnp.concatenate([out0[:N_NODES], out1[:N_NODES]], axis=1)
    return final[:U_NUM], final[U_NUM:]


# speedup vs baseline: 1.0390x; 1.0390x over previous
import functools

import jax
import jax.numpy as jnp
from jax import lax
from jax.experimental import pallas as pl
from jax.experimental.pallas import tpu as pltpu
from jax.experimental.pallas import tpu_sc as plsc

U_NUM = 5000
I_NUM = 5000
N_NODES = U_NUM + I_NUM
DIM = 128
HALF = 64
LAYERS = 3
N_EDGES = 320000

NC = 2
NS = 16
GRP = 128

N_PAD = 10240
E_PAD = 327680
GROUPS = E_PAD // (NS * GRP)
RPT = N_PAD // NS
RSUB = 64
NSUB = RPT // RSUB


def _sc_body(h0, h1, srcr, dstr, valr,
             out0, out1,
             spa, spb, valv, rows0, rows1, bufa, bufb,
             srcs, dsts,
             semg0, semg1, sems0, sems1, semi0, semi1):
    c = lax.axis_index("c")
    s = lax.axis_index("s")
    src0, src1 = srcs.at[0], srcs.at[1]
    dst0, dst1 = dsts.at[0], dsts.at[1]

    pltpu.sync_copy(valr.at[s], valv)

    zero16 = jnp.zeros((16,), jnp.float32)

    def zero_buf(buf, nrows):
        @pl.loop(0, nrows)
        def _(i):
            for q in range(4):
                buf[i, pl.ds(q * 16, 16)] = zero16

    zero_buf(bufb, RSUB)

    @pl.loop(0, NSUB)
    def _(r):
        sl = pl.ds(s * RPT + r * RSUB, RSUB)

        @pl.when(c == 0)
        def _():
            pltpu.sync_copy(h0.at[sl], bufa)
            pltpu.sync_copy(bufa, out0.at[sl])

        @pl.when(c == 1)
        def _():
            pltpu.sync_copy(h1.at[sl], bufa)
            pltpu.sync_copy(bufa, out1.at[sl])

        pltpu.sync_copy(bufa, spa.at[sl])
        pltpu.sync_copy(bufb, spb.at[sl])

    plsc.subcore_barrier()

    def stage_idx(gg, sbuf, dbuf, sm):
        pltpu.async_copy(srcr.at[s].at[gg], sbuf, sm)
        pltpu.async_copy(dstr.at[s].at[gg], dbuf, sm)

    def idx_wait(sbuf, dbuf, sm):
        pltpu.make_async_copy(srcr.at[s].at[0], sbuf, sm).wait()
        pltpu.make_async_copy(dstr.at[s].at[0], dbuf, sm).wait()

    def scale(buf, gg):
        gbase = gg * GRP

        @plsc.parallel_loop(0, GRP, unroll=8)
        def _(i):
            sp = plsc.load_gather(valv, [jnp.full((16,), gbase + i, jnp.int32)])
            for q in range(4):
                buf[i, pl.ds(q * 16, 16)] = buf[i, pl.ds(q * 16, 16)] * sp

    for l in range(LAYERS):
        last = l == LAYERS - 1
        srcsp = spa if l % 2 == 0 else spb
        dstsp = spb if l % 2 == 0 else spa

        def gather_start(buf, sbuf, sm):
            pltpu.async_copy(srcsp.at[sbuf], buf, sm)

        def gather_wait(buf, sbuf, sm):
            pltpu.make_async_copy(srcsp.at[sbuf], buf, sm).wait()

        def scatter_start(buf, dbuf, sm):
            pltpu.async_copy(buf, dstsp.at[dbuf], sm, add=True)

        def scatter_wait(buf, dbuf, sm):
            pltpu.make_async_copy(buf, dstsp.at[dbuf], sm).wait()

        stage_idx(0, src0, dst0, semi0)
        idx_wait(src0, dst0, semi0)
        gather_start(rows0, src0, semg0)

        @pl.loop(0, GROUPS, step=2)
        def _(g):
            @pl.when(g > 0)
            def _():
                scatter_wait(rows1, dst1, sems1)
            stage_idx(g + 1, src1, dst1, semi1)
            idx_wait(src1, dst1, semi1)
            gather_start(rows1, src1, semg1)
            gather_wait(rows0, src0, semg0)
            scale(rows0, g)
            scatter_start(rows0, dst0, sems0)

            gather_wait(rows1, src1, semg1)
            scatter_wait(rows0, dst0, sems0)

            @pl.when(g + 2 < GROUPS)
            def _():
                stage_idx(g + 2, src0, dst0, semi0)
                idx_wait(src0, dst0, semi0)
                gather_start(rows0, src0, semg0)

            scale(rows1, g + 1)
            scatter_start(rows1, dst1, sems1)

        scatter_wait(rows1, dst1, sems1)
        plsc.subcore_barrier()

        @pl.loop(0, NSUB)
        def _(r):
            sl = pl.ds(s * RPT + r * RSUB, RSUB)
            pltpu.sync_copy(dstsp.at[sl], bufa)

            @pl.when(c == 0)
            def _():
                pltpu.sync_copy(out0.at[sl], bufb)

            @pl.when(c == 1)
            def _():
                pltpu.sync_copy(out1.at[sl], bufb)

            @pl.loop(0, RSUB)
            def _(i):
                for q in range(4):
                    v = bufa[i, pl.ds(q * 16, 16)] + bufb[i, pl.ds(q * 16, 16)]
                    if last:
                        v = v * jnp.float32(1.0 / (LAYERS + 1))
                    bufb[i, pl.ds(q * 16, 16)] = v

            @pl.when(c == 0)
            def _():
                pltpu.sync_copy(bufb, out0.at[sl])

            @pl.when(c == 1)
            def _():
                pltpu.sync_copy(bufb, out1.at[sl])

            if not last:
                zero_buf(bufa, RSUB)
                pltpu.sync_copy(bufa, srcsp.at[sl])

        if not last:
            plsc.subcore_barrier()


@functools.partial(
    pl.kernel,
    out_type=(
        jax.ShapeDtypeStruct((N_PAD, HALF), jnp.float32),
        jax.ShapeDtypeStruct((N_PAD, HALF), jnp.float32),
    ),
    mesh=plsc.VectorSubcoreMesh(
        core_axis_name="c", subcore_axis_name="s", num_cores=NC, num_subcores=NS
    ),
    compiler_params=pltpu.CompilerParams(
        needs_layout_passes=False, use_tc_tiling_on_sc=False
    ),
    scratch_types=[
        pltpu.VMEM_SHARED((N_PAD, HALF), jnp.float32),
        pltpu.VMEM_SHARED((N_PAD, HALF), jnp.float32),
        pltpu.VMEM((GROUPS * GRP,), jnp.float32),
        pltpu.VMEM((GRP, HALF), jnp.float32),
        pltpu.VMEM((GRP, HALF), jnp.float32),
        pltpu.VMEM((RSUB, HALF), jnp.float32),
        pltpu.VMEM((RSUB, HALF), jnp.float32),
        pltpu.VMEM((2, GRP), jnp.int32),
        pltpu.VMEM((2, GRP), jnp.int32),
        pltpu.SemaphoreType.DMA,
        pltpu.SemaphoreType.DMA,
        pltpu.SemaphoreType.DMA,
        pltpu.SemaphoreType.DMA,
        pltpu.SemaphoreType.DMA,
        pltpu.SemaphoreType.DMA,
    ],
)
def _lightgcn_sc(h0, h1, srcr, dstr, valr, out0, out1,
                 spa, spb, valv, rows0, rows1, bufa, bufb,
                 srcs, dsts,
                 semg0, semg1, sems0, sems1, semi0, semi1):
    _sc_body(h0, h1, srcr, dstr, valr, out0, out1,
             spa, spb, valv, rows0, rows1, bufa, bufb,
             srcs, dsts,
             semg0, semg1, sems0, sems1, semi0, semi1)


def kernel(user_embeds, item_embeds, adj_values, adj_indices, keep_rate):
    del keep_rate
    f32 = jnp.float32

    h0 = jnp.zeros((N_PAD, HALF), f32)
    h0 = h0.at[:U_NUM].set(user_embeds[:, :HALF].astype(f32))
    h0 = h0.at[U_NUM:N_NODES].set(item_embeds[:, :HALF].astype(f32))
    h1 = jnp.zeros((N_PAD, HALF), f32)
    h1 = h1.at[:U_NUM].set(user_embeds[:, HALF:].astype(f32))
    h1 = h1.at[U_NUM:N_NODES].set(item_embeds[:, HALF:].astype(f32))

    pad = E_PAD - N_EDGES
    src = jnp.concatenate(
        [adj_indices[1].astype(jnp.int32), jnp.zeros((pad,), jnp.int32)]
    ).reshape(NS, GROUPS, GRP)
    dst = jnp.concatenate(
        [adj_indices[0].astype(jnp.int32), jnp.zeros((pad,), jnp.int32)]
    ).reshape(NS, GROUPS, GRP)
    val = jnp.concatenate(
        [adj_values.astype(f32), jnp.zeros((pad,), f32)]
    ).reshape(NS, GROUPS * GRP)

    out0, out1 = _lightgcn_sc(h0, h1, src, dst, val)
    final = jnp.concatenate([out0[:N_NODES], out1[:N_NODES]], axis=1)
    return final[:U_NUM], final[U_NUM:]


# speedup vs baseline: 1.2763x; 1.2285x over previous
import functools

import jax
import jax.numpy as jnp
from jax import lax
from jax.experimental import pallas as pl
from jax.experimental.pallas import tpu as pltpu
from jax.experimental.pallas import tpu_sc as plsc

U_NUM = 5000
I_NUM = 5000
N_NODES = U_NUM + I_NUM
DIM = 128
HALF = 64
LAYERS = 3
N_EDGES = 320000

NC = 2
NS = 16
GRP = 128

N_PAD = 10240
E_PAD = 327680
GROUPS = E_PAD // (NS * GRP)
RPT = N_PAD // NS
RSUB = 64
NSUB = RPT // RSUB


def _sc_body(h0, h1, srcr, dstr, valr,
             out0, out1,
             spa, spb, valv, rows0, rows1, bufa, bufb,
             srcs, dsts,
             semg0, semg1, sems0, sems1, semi0, semi1, semi2, semi3):
    c = lax.axis_index("c")
    s = lax.axis_index("s")
    srcsl = [srcs.at[b] for b in range(4)]
    dstsl = [dsts.at[b] for b in range(4)]
    semis = [semi0, semi1, semi2, semi3]

    pltpu.sync_copy(valr.at[s], valv)

    zero16 = jnp.zeros((16,), jnp.float32)

    def zero_buf(buf, nrows):
        @pl.loop(0, nrows)
        def _(i):
            for q in range(4):
                buf[i, pl.ds(q * 16, 16)] = zero16

    zero_buf(bufb, RSUB)

    @pl.loop(0, NSUB)
    def _(r):
        sl = pl.ds(s * RPT + r * RSUB, RSUB)

        @pl.when(c == 0)
        def _():
            pltpu.sync_copy(h0.at[sl], bufa)
            pltpu.sync_copy(bufa, out0.at[sl])

        @pl.when(c == 1)
        def _():
            pltpu.sync_copy(h1.at[sl], bufa)
            pltpu.sync_copy(bufa, out1.at[sl])

        pltpu.sync_copy(bufa, spa.at[sl])
        pltpu.sync_copy(bufb, spb.at[sl])

    plsc.subcore_barrier()

    def stage_idx(gg, b):
        pltpu.async_copy(srcr.at[s].at[gg], srcsl[b], semis[b])
        pltpu.async_copy(dstr.at[s].at[gg], dstsl[b], semis[b])

    def idx_wait(b):
        pltpu.make_async_copy(srcr.at[s].at[0], srcsl[b], semis[b]).wait()
        pltpu.make_async_copy(dstr.at[s].at[0], dstsl[b], semis[b]).wait()

    def scale(buf, gg):
        gbase = gg * GRP

        @plsc.parallel_loop(0, GRP, unroll=8)
        def _(i):
            sp = plsc.load_gather(valv, [jnp.full((16,), gbase + i, jnp.int32)])
            for q in range(4):
                buf[i, pl.ds(q * 16, 16)] = buf[i, pl.ds(q * 16, 16)] * sp

    for l in range(LAYERS):
        last = l == LAYERS - 1
        srcsp = spa if l % 2 == 0 else spb
        dstsp = spb if l % 2 == 0 else spa

        def gather_start(buf, b, sm):
            pltpu.async_copy(srcsp.at[srcsl[b]], buf, sm)

        def gather_wait(buf, b, sm):
            pltpu.make_async_copy(srcsp.at[srcsl[b]], buf, sm).wait()

        def scatter_start(buf, b, sm):
            pltpu.async_copy(buf, dstsp.at[dstsl[b]], sm, add=True)

        def scatter_wait(buf, b, sm):
            pltpu.make_async_copy(buf, dstsp.at[dstsl[b]], sm).wait()

        stage_idx(0, 0)
        stage_idx(1, 1)
        idx_wait(0)
        gather_start(rows0, 0, semg0)

        @pl.loop(0, GROUPS, step=4)
        def _(g):
            @pl.when(g > 0)
            def _():
                scatter_wait(rows1, 3, sems1)
            stage_idx(g + 2, 2)
            stage_idx(g + 3, 3)
            idx_wait(1)
            gather_start(rows1, 1, semg1)
            gather_wait(rows0, 0, semg0)
            scale(rows0, g)
            scatter_start(rows0, 0, sems0)

            gather_wait(rows1, 1, semg1)
            scatter_wait(rows0, 0, sems0)
            idx_wait(2)
            gather_start(rows0, 2, semg0)
            scale(rows1, g + 1)
            scatter_start(rows1, 1, sems1)

            scatter_wait(rows1, 1, sems1)

            @pl.when(g + 5 < GROUPS)
            def _():
                stage_idx(g + 4, 0)
                stage_idx(g + 5, 1)

            idx_wait(3)
            gather_start(rows1, 3, semg1)
            gather_wait(rows0, 2, semg0)
            scale(rows0, g + 2)
            scatter_start(rows0, 2, sems0)

            gather_wait(rows1, 3, semg1)
            scatter_wait(rows0, 2, sems0)

            @pl.when(g + 4 < GROUPS)
            def _():
                idx_wait(0)
                gather_start(rows0, 0, semg0)

            scale(rows1, g + 3)
            scatter_start(rows1, 3, sems1)

        scatter_wait(rows1, 3, sems1)
        plsc.subcore_barrier()

        @pl.loop(0, NSUB)
        def _(r):
            sl = pl.ds(s * RPT + r * RSUB, RSUB)
            pltpu.sync_copy(dstsp.at[sl], bufa)

            @pl.when(c == 0)
            def _():
                pltpu.sync_copy(out0.at[sl], bufb)

            @pl.when(c == 1)
            def _():
                pltpu.sync_copy(out1.at[sl], bufb)

            @pl.loop(0, RSUB)
            def _(i):
                for q in range(4):
                    v = bufa[i, pl.ds(q * 16, 16)] + bufb[i, pl.ds(q * 16, 16)]
                    if last:
                        v = v * jnp.float32(1.0 / (LAYERS + 1))
                    bufb[i, pl.ds(q * 16, 16)] = v

            @pl.when(c == 0)
            def _():
                pltpu.sync_copy(bufb, out0.at[sl])

            @pl.when(c == 1)
            def _():
                pltpu.sync_copy(bufb, out1.at[sl])

            if not last:
                zero_buf(bufa, RSUB)
                pltpu.sync_copy(bufa, srcsp.at[sl])

        if not last:
            plsc.subcore_barrier()


@functools.partial(
    pl.kernel,
    out_type=(
        jax.ShapeDtypeStruct((N_PAD, HALF), jnp.float32),
        jax.ShapeDtypeStruct((N_PAD, HALF), jnp.float32),
    ),
    mesh=plsc.VectorSubcoreMesh(
        core_axis_name="c", subcore_axis_name="s", num_cores=NC, num_subcores=NS
    ),
    compiler_params=pltpu.CompilerParams(
        needs_layout_passes=False, use_tc_tiling_on_sc=False
    ),
    scratch_types=[
        pltpu.VMEM_SHARED((N_PAD, HALF), jnp.float32),
        pltpu.VMEM_SHARED((N_PAD, HALF), jnp.float32),
        pltpu.VMEM((GROUPS * GRP,), jnp.float32),
        pltpu.VMEM((GRP, HALF), jnp.float32),
        pltpu.VMEM((GRP, HALF), jnp.float32),
        pltpu.VMEM((RSUB, HALF), jnp.float32),
        pltpu.VMEM((RSUB, HALF), jnp.float32),
        pltpu.VMEM((4, GRP), jnp.int32),
        pltpu.VMEM((4, GRP), jnp.int32),
        pltpu.SemaphoreType.DMA,
        pltpu.SemaphoreType.DMA,
        pltpu.SemaphoreType.DMA,
        pltpu.SemaphoreType.DMA,
        pltpu.SemaphoreType.DMA,
        pltpu.SemaphoreType.DMA,
        pltpu.SemaphoreType.DMA,
        pltpu.SemaphoreType.DMA,
    ],
)
def _lightgcn_sc(h0, h1, srcr, dstr, valr, out0, out1,
                 spa, spb, valv, rows0, rows1, bufa, bufb,
                 srcs, dsts,
                 semg0, semg1, sems0, sems1, semi0, semi1, semi2, semi3):
    _sc_body(h0, h1, srcr, dstr, valr, out0, out1,
             spa, spb, valv, rows0, rows1, bufa, bufb,
             srcs, dsts,
             semg0, semg1, sems0, sems1, semi0, semi1, semi2, semi3)


def kernel(user_embeds, item_embeds, adj_values, adj_indices, keep_rate):
    del keep_rate
    f32 = jnp.float32

    h0 = jnp.zeros((N_PAD, HALF), f32)
    h0 = h0.at[:U_NUM].set(user_embeds[:, :HALF].astype(f32))
    h0 = h0.at[U_NUM:N_NODES].set(item_embeds[:, :HALF].astype(f32))
    h1 = jnp.zeros((N_PAD, HALF), f32)
    h1 = h1.at[:U_NUM].set(user_embeds[:, HALF:].astype(f32))
    h1 = h1.at[U_NUM:N_NODES].set(item_embeds[:, HALF:].astype(f32))

    pad = E_PAD - N_EDGES
    src = jnp.concatenate(
        [adj_indices[1].astype(jnp.int32), jnp.zeros((pad,), jnp.int32)]
    ).reshape(NS, GROUPS, GRP)
    dst = jnp.concatenate(
        [adj_indices[0].astype(jnp.int32), jnp.zeros((pad,), jnp.int32)]
    ).reshape(NS, GROUPS, GRP)
    val = jnp.concatenate(
        [adj_values.astype(f32), jnp.zeros((pad,), f32)]
    ).reshape(NS, GROUPS * GRP)

    out0, out1 = _lightgcn_sc(h0, h1, src, dst, val)
    final = jnp.concatenate([out0[:N_NODES], out1[:N_NODES]], axis=1)
    return final[:U_NUM], final[U_NUM:]
